# baseline (speedup 1.0000x reference)
import functools

import jax
import jax.numpy as jnp
from jax import lax
from jax.experimental import pallas as pl
from jax.experimental.pallas import tpu as pltpu
from jax.experimental.pallas import tpu_sc as plsc

_CHUNK = 32
_NW = 32


def _bcast_sc(pe_flat, B, L, D):
    n_chunks = -(-L // _CHUNK)
    per_w = n_chunks // _NW
    tail = (L - (n_chunks - 1) * _CHUNK) * D
    cs = _CHUNK * D
    mesh = plsc.VectorSubcoreMesh(core_axis_name="c", subcore_axis_name="s")

    @functools.partial(
        pl.kernel,
        out_type=jax.ShapeDtypeStruct((B * L * D,), jnp.float32),
        mesh=mesh,
        scratch_types=[pltpu.VMEM((cs,), jnp.float32)],
    )
    def body(pe_hbm, out_hbm, buf):
        w = lax.axis_index("c") * 16 + lax.axis_index("s")

        def full(ci):
            off = ci * cs
            pltpu.sync_copy(pe_hbm.at[pl.ds(off, cs)], buf)
            for b in range(B):
                pltpu.sync_copy(buf, out_hbm.at[pl.ds(b * L * D + off, cs)])

        for k in range(per_w - 1):
            full(w + _NW * k)

        @pl.when(w != _NW - 1)
        def _():
            full(w + _NW * (per_w - 1))

        @pl.when(w == _NW - 1)
        def _():
            off = (n_chunks - 1) * cs
            pltpu.sync_copy(pe_hbm.at[pl.ds(off, tail)], buf.at[pl.ds(0, tail)])
            for b in range(B):
                pltpu.sync_copy(buf.at[pl.ds(0, tail)],
                                out_hbm.at[pl.ds(b * L * D + off, tail)])

    return body(pe_flat)


def kernel(x, pe):
    B, S, D = x.shape
    L = 2 * S - 1
    out = _bcast_sc(pe.reshape(-1), B, L, D)
    return out.reshape(B, L, D)


# speedup vs baseline: 3.8045x; 3.8045x over previous
import functools

import jax
import jax.numpy as jnp
from jax.experimental import pallas as pl
from jax.experimental.pallas import tpu as pltpu

_CH = 512


def _bcast_tc(pe3d, B, L, D):
    n_chunks = -(-L // _CH)

    def body(pe_ref, out_ref):
        out_ref[...] = pe_ref[...]

    return pl.pallas_call(
        body,
        grid=(n_chunks, B),
        in_specs=[pl.BlockSpec((1, _CH, D), lambda i, b: (0, i, 0))],
        out_specs=pl.BlockSpec((1, _CH, D), lambda i, b: (b, i, 0)),
        out_shape=jax.ShapeDtypeStruct((B, L, D), jnp.float32),
        compiler_params=pltpu.CompilerParams(
            dimension_semantics=("arbitrary", "arbitrary"),
        ),
    )(pe3d)


def kernel(x, pe):
    B, S, D = x.shape
    L = 2 * S - 1
    return _bcast_tc(pe, B, L, D)


# speedup vs baseline: 4.2808x; 1.1252x over previous
import functools

import jax
import jax.numpy as jnp
from jax.experimental import pallas as pl
from jax.experimental.pallas import tpu as pltpu

_CH = 512


def _bcast_tc(pe3d, B, L, D):
    n_chunks = -(-L // _CH)

    def body(pe_ref, out_ref):
        blk = pe_ref[...]
        out_ref[...] = jnp.broadcast_to(blk, (B, _CH, D))

    return pl.pallas_call(
        body,
        grid=(n_chunks,),
        in_specs=[pl.BlockSpec((1, _CH, D), lambda i: (0, i, 0))],
        out_specs=pl.BlockSpec((B, _CH, D), lambda i: (0, i, 0)),
        out_shape=jax.ShapeDtypeStruct((B, L, D), jnp.float32),
        compiler_params=pltpu.CompilerParams(
            dimension_semantics=("arbitrary",),
        ),
    )(pe3d)


def kernel(x, pe):
    B, S, D = x.shape
    L = 2 * S - 1
    return _bcast_tc(pe, B, L, D)
